# baseline (speedup 1.0000x reference)
import functools

import jax
import jax.numpy as jnp
from jax import lax
from jax.experimental import pallas as pl
from jax.experimental.pallas import tpu as pltpu
from jax.experimental.pallas import tpu_sc as plsc

_NC = 2
_NS = 16
_NW = _NC * _NS
_CHUNK = 128


def _pass_a_body(w0_ref, x_ref, b_ref, e_ref, o_ref, w_scr):
    i = pl.program_id(0)
    d = w_scr.shape[0]

    @pl.when(i == 0)
    def _():
        w_scr[...] = jnp.concatenate(
            [w0_ref[pl.ds(k * d, d), :] for k in range(8)],
            axis=1).astype(jnp.bfloat16)

    @pl.when(i > 0)
    def _():
        xb = x_ref[...].astype(jnp.bfloat16)
        acc = jnp.dot(xb, w_scr[...], preferred_element_type=jnp.float32)
        h = jnp.maximum(acc + b_ref[...], 0.0)
        ev = e_ref[...]
        sel = h[:, :64] * (ev == 0.0)
        for k in range(1, 8):
            sel = sel + h[:, 64 * k:64 * (k + 1)] * (ev == float(k))
        o_ref[:, :64] = sel
        o_ref[:, 64:] = sel


def _pass_a(state, w0flat, b0all, e2d, block_rows=512):
    b, d = state.shape
    ed, h = w0flat.shape
    e = ed // d
    eh = e * h

    def shifted(i):
        return jnp.maximum(i - 1, 0)

    return pl.pallas_call(
        _pass_a_body,
        grid=(b // block_rows + 1,),
        in_specs=[
            pl.BlockSpec((ed, h), lambda i: (0, 0)),
            pl.BlockSpec((block_rows, d), lambda i: (shifted(i), 0)),
            pl.BlockSpec((1, eh), lambda i: (0, 0)),
            pl.BlockSpec((block_rows, 1), lambda i: (shifted(i), 0)),
        ],
        out_specs=pl.BlockSpec((block_rows, 128), lambda i: (shifted(i), 0)),
        out_shape=jax.ShapeDtypeStruct((b, 128), jnp.float32),
        scratch_shapes=[pltpu.VMEM((d, eh), jnp.bfloat16)],
    )(w0flat, state, b0all, e2d)


def _sc_dispatch(h0sel, idx_dst, p_rows):
    nw, k, c = idx_dst.shape
    per_w = k * c
    width = h0sel.shape[1]
    mesh = plsc.VectorSubcoreMesh(core_axis_name="c", subcore_axis_name="s")

    @functools.partial(
        pl.kernel,
        mesh=mesh,
        out_type=jax.ShapeDtypeStruct((p_rows, width), jnp.float32),
        scratch_types=[
            pltpu.VMEM((k, c), jnp.int32),
            pltpu.VMEM((per_w, width), jnp.float32),
            pltpu.SemaphoreType.DMA,
        ],
    )
    def kern(h0_hbm, idst_hbm, xpad_hbm, idst_v, rows_v, sem):
        wid = lax.axis_index("s") * _NC + lax.axis_index("c")
        pltpu.sync_copy(idst_hbm.at[wid], idst_v)
        pltpu.sync_copy(h0_hbm.at[pl.ds(wid * per_w, per_w)], rows_v)
        scatters = []
        for j in range(k):
            scatters.append(pltpu.async_copy(
                rows_v.at[pl.ds(j * c, c)],
                xpad_hbm.at[idst_v.at[j]], sem))
        for s in scatters:
            s.wait()

    return kern(h0sel, idx_dst)


def _sc_collect(y_pad, idx, b_rows, a):
    nw, k, c = idx.shape
    per_w = k * c
    width = y_pad.shape[1]
    mesh = plsc.VectorSubcoreMesh(core_axis_name="c", subcore_axis_name="s")

    @functools.partial(
        pl.kernel,
        mesh=mesh,
        out_type=jax.ShapeDtypeStruct((b_rows, width), jnp.float32),
        scratch_types=[
            pltpu.VMEM((k, c), jnp.int32),
            pltpu.VMEM((per_w, width), jnp.float32),
            pltpu.SemaphoreType.DMA,
        ],
    )
    def kern(ypad_hbm, idx_hbm, out_hbm, idx_v, rows_v, sem):
        wid = lax.axis_index("s") * _NC + lax.axis_index("c")
        pltpu.sync_copy(idx_hbm.at[wid], idx_v)
        gathers = []
        for j in range(k):
            gathers.append(pltpu.async_copy(
                ypad_hbm.at[idx_v.at[j]],
                rows_v.at[pl.ds(j * c, c)], sem))
        for g in gathers:
            g.wait()
        pltpu.sync_copy(rows_v, out_hbm.at[pl.ds(wid * per_w, per_w)])

    return kern(y_pad, idx)


def _pass_b_body(x_ref, wt_ref, bt_ref, o_ref):
    zero = jnp.zeros((64, 64), jnp.bfloat16)

    def bd(l):
        rows = []
        for q in range(4):
            pieces = [zero] * 4
            pieces[q] = wt_ref[l, q]
            rows.append(jnp.concatenate(pieces, axis=1))
        return jnp.concatenate(rows, axis=0)

    def bias(l):
        return jnp.concatenate([bt_ref[q, l, :] for q in range(4)])

    x4 = jnp.concatenate(
        [x_ref[q][:, :64] for q in range(4)], axis=1)
    h = x4.astype(jnp.bfloat16)
    for l in range(4):
        acc = jnp.dot(h, bd(l), preferred_element_type=jnp.float32)
        h = jnp.maximum(acc + bias(l), 0.0).astype(jnp.bfloat16)
    y4 = (jnp.dot(h, bd(4), preferred_element_type=jnp.float32)
          + bias(4))
    for q in range(4):
        o_ref[q, :, :64] = y4[:, 64 * q:64 * (q + 1)]
        o_ref[q, :, 64:] = y4[:, 64 * q:64 * (q + 1)]


def _pass_b(x_pad4, wt, bt, tile_rows, n_tiles, h, a):
    nq = n_tiles // 4
    return pl.pallas_call(
        _pass_b_body,
        grid=(nq,),
        in_specs=[
            pl.BlockSpec((4, tile_rows, 128), lambda t: (t, 0, 0)),
            pl.BlockSpec((5, 4, h, h), lambda t: (0, t, 0, 0)),
            pl.BlockSpec((4, 5, a), lambda t: (t, 0, 0)),
        ],
        out_specs=pl.BlockSpec((4, tile_rows, 128), lambda t: (t, 0, 0)),
        out_shape=jax.ShapeDtypeStruct((n_tiles, tile_rows, 128),
                                       jnp.float32),
    )(x_pad4, wt, bt)


def kernel(state, rm_state, W0, b0, W1, b1, W2, b2, W3, b3, W4, b4, W5, b5):
    B, D = state.shape
    E, _, H = W0.shape
    A = W5.shape[2]
    T = 256
    NT = B // T + E
    P = NT * T

    e = rm_state.astype(jnp.int32)
    oh = (e[:, None] == jnp.arange(E, dtype=jnp.int32)[None, :]).astype(jnp.int32)
    cs = jnp.cumsum(oh, axis=0)
    cnt = cs[-1]
    occ = jnp.sum((cs - oh) * oh, axis=1)
    tiles_e = (cnt + T - 1) // T
    tile_start = jnp.concatenate(
        [jnp.zeros((1,), jnp.int32), jnp.cumsum(tiles_e)[:-1].astype(jnp.int32)])
    row_start = tile_start * T
    p = jnp.sum(oh * row_start[None, :], axis=1) + occ
    idx_dst = p.reshape(_NW, -1, _CHUNK)
    tile_expert = (jnp.sum(
        (jnp.arange(NT, dtype=jnp.int32)[:, None] >= tile_start[None, :])
        .astype(jnp.int32), axis=1) - 1)
    e2d = e.astype(jnp.float32).reshape(B, 1)

    wstk = jnp.stack((W1, W2, W3, W4, W5)).astype(jnp.bfloat16)
    wt = wstk[:, tile_expert]
    bstack = jnp.stack((b1, b2, b3, b4, b5), axis=1)
    bt = bstack[tile_expert]

    b0all = b0.reshape(1, E * H)
    h0sel = _pass_a(state, W0.reshape(E * D, H), b0all, e2d)
    x_pad = _sc_dispatch(h0sel, idx_dst, P)
    x_pad4 = x_pad.reshape(NT, T, 128)
    y_pad4 = _pass_b(x_pad4, wt, bt, T, NT, H, A)
    y_pad = y_pad4.reshape(P, 128)
    wide = _sc_collect(y_pad, idx_dst, B, A)
    return wide[:, :A]


# speedup vs baseline: 1.1608x; 1.1608x over previous
import functools

import jax
import jax.numpy as jnp
from jax import lax
from jax.experimental import pallas as pl
from jax.experimental.pallas import tpu as pltpu
from jax.experimental.pallas import tpu_sc as plsc

_NC = 2
_NS = 16
_NW = _NC * _NS
_CHUNK = 128


def _pass_a_body(w0_ref, x_ref, b_ref, e_ref, o_ref, w_scr):
    i = pl.program_id(0)
    d = w_scr.shape[0]

    @pl.when(i == 0)
    def _():
        w_scr[...] = jnp.concatenate(
            [w0_ref[pl.ds(k * d, d), :] for k in range(8)],
            axis=1).astype(jnp.bfloat16)

    @pl.when(i > 0)
    def _():
        xb = x_ref[...].astype(jnp.bfloat16)
        acc = jnp.dot(xb, w_scr[...], preferred_element_type=jnp.float32)
        h = jnp.maximum(acc + b_ref[...], 0.0)
        ev = e_ref[...]
        sel = h[:, :64] * (ev == 0.0)
        for k in range(1, 8):
            sel = sel + h[:, 64 * k:64 * (k + 1)] * (ev == float(k))
        o_ref[:, :64] = sel
        o_ref[:, 64:] = sel


def _pass_a(state, w0flat, b0all, e2d, block_rows=512):
    b, d = state.shape
    ed, h = w0flat.shape
    e = ed // d
    eh = e * h

    def shifted(i):
        return jnp.maximum(i - 1, 0)

    return pl.pallas_call(
        _pass_a_body,
        grid=(b // block_rows + 1,),
        in_specs=[
            pl.BlockSpec((ed, h), lambda i: (0, 0)),
            pl.BlockSpec((block_rows, d), lambda i: (shifted(i), 0)),
            pl.BlockSpec((1, eh), lambda i: (0, 0)),
            pl.BlockSpec((block_rows, 1), lambda i: (shifted(i), 0)),
        ],
        out_specs=pl.BlockSpec((block_rows, 128), lambda i: (shifted(i), 0)),
        out_shape=jax.ShapeDtypeStruct((b, 128), jnp.float32),
        scratch_shapes=[pltpu.VMEM((d, eh), jnp.bfloat16)],
    )(w0flat, state, b0all, e2d)


def _sc_route(e_arr, b_rows, t_rows, nt, nw, chunk):
    mesh = plsc.VectorSubcoreMesh(core_axis_name="c", subcore_axis_name="s", num_cores=_NC, num_subcores=_NS)
    per_s = b_rows // _NS
    k = per_s // (2 * chunk)
    nch = per_s // 16

    @functools.partial(
        pl.kernel,
        mesh=mesh,
        compiler_params=pltpu.CompilerParams(needs_layout_passes=False),
        out_type=[
            jax.ShapeDtypeStruct((nw, per_s // (chunk * 2), chunk), jnp.int32),
            jax.ShapeDtypeStruct((32,), jnp.int32),
            jax.ShapeDtypeStruct((_NS, 16), jnp.int32),
        ],
        scratch_types=[
            pltpu.VMEM((per_s,), jnp.int32),
            pltpu.VMEM((per_s,), jnp.int32),
            pltpu.VMEM((16,), jnp.int32),
            pltpu.VMEM((_NS, 16), jnp.int32),
            pltpu.VMEM((16,), jnp.int32),
            pltpu.VMEM((16,), jnp.int32),
            pltpu.VMEM((2, 2, chunk), jnp.int32),
            pltpu.VMEM((32,), jnp.int32),
        ],
    )
    def kern(e_hbm, idst_hbm, te_hbm, cnt_hbm,
             e_v, occ_v, cnt_v, all_v, rsb_v, ts_v, p_v, te_v):
        cid = lax.axis_index("c")
        sid = lax.axis_index("s")

        @pl.when(cid == 0)
        def _():
            iota = lax.iota(jnp.int32, 16)
            pltpu.sync_copy(e_hbm.at[pl.ds(sid * per_s, per_s)], e_v)
            cnt_v[...] = jnp.zeros(16, jnp.int32)
            for ch in range(nch):
                ev = e_v[pl.ds(ch * 16, 16)]
                base = plsc.load_gather(cnt_v, [ev])
                cnt = cnt_v[...]
                within = jnp.zeros(16, jnp.int32)
                for ex in range(8):
                    m = (ev == ex).astype(jnp.int32)
                    pre = plsc.cumsum(m) - m
                    within = within + pre * m
                    tot = jnp.max(plsc.cumsum(m))
                    cnt = cnt + jnp.where(iota == ex, tot, 0)
                occ_v[pl.ds(ch * 16, 16)] = base + within
                cnt_v[...] = cnt
            pltpu.sync_copy(cnt_v, cnt_hbm.at[sid])
            plsc.subcore_barrier()
            pltpu.sync_copy(cnt_hbm, all_v)
            base = jnp.zeros(16, jnp.int32)
            gcnt = jnp.zeros(16, jnp.int32)
            for s in range(_NS):
                row = all_v[s]
                below = jnp.full((16,), s, jnp.int32) < sid
                base = base + jnp.where(below, row, 0)
                gcnt = gcnt + row
            tiles = (gcnt + (t_rows - 1)) // t_rows
            ts_incl = plsc.cumsum(tiles)
            row_start = (ts_incl - tiles) * t_rows
            rsb_v[...] = row_start + base
            ts_v[...] = ts_incl - tiles
            for ch in range(nch):
                ev = e_v[pl.ds(ch * 16, 16)]
                slot = (plsc.load_gather(rsb_v, [ev])
                        + occ_v[pl.ds(ch * 16, 16)])
                p_v[ch // (nch // 2), (ch % (nch // 2)) // (chunk // 16),
                    pl.ds((ch * 16) % chunk, 16)] = slot
            pltpu.sync_copy(p_v, idst_hbm.at[pl.ds(2 * sid, 2)])
            te_v[pl.ds(0, 16)] = ts_v[...]
            pltpu.sync_copy(te_v, te_hbm)

    return kern(e_arr)


def _sc_dispatch(h0sel, idx_dst, p_rows):
    nw, k, c = idx_dst.shape
    per_w = k * c
    width = h0sel.shape[1]
    mesh = plsc.VectorSubcoreMesh(core_axis_name="c", subcore_axis_name="s", num_cores=_NC, num_subcores=_NS)

    @functools.partial(
        pl.kernel,
        mesh=mesh,
        out_type=jax.ShapeDtypeStruct((p_rows, width), jnp.float32),
        scratch_types=[
            pltpu.VMEM((k, c), jnp.int32),
            pltpu.VMEM((per_w, width), jnp.float32),
            pltpu.SemaphoreType.DMA,
        ],
    )
    def kern(h0_hbm, idst_hbm, xpad_hbm, idst_v, rows_v, sem):
        wid = lax.axis_index("s") * _NC + lax.axis_index("c")
        pltpu.sync_copy(idst_hbm.at[wid], idst_v)
        pltpu.sync_copy(h0_hbm.at[pl.ds(wid * per_w, per_w)], rows_v)
        scatters = []
        for j in range(k):
            scatters.append(pltpu.async_copy(
                rows_v.at[pl.ds(j * c, c)],
                xpad_hbm.at[idst_v.at[j]], sem))
        for s in scatters:
            s.wait()

    return kern(h0sel, idx_dst)


def _sc_collect(y_pad, idx, b_rows, a):
    nw, k, c = idx.shape
    per_w = k * c
    width = y_pad.shape[1]
    mesh = plsc.VectorSubcoreMesh(core_axis_name="c", subcore_axis_name="s", num_cores=_NC, num_subcores=_NS)

    @functools.partial(
        pl.kernel,
        mesh=mesh,
        out_type=jax.ShapeDtypeStruct((b_rows, width), jnp.float32),
        scratch_types=[
            pltpu.VMEM((k, c), jnp.int32),
            pltpu.VMEM((per_w, width), jnp.float32),
            pltpu.SemaphoreType.DMA,
        ],
    )
    def kern(ypad_hbm, idx_hbm, out_hbm, idx_v, rows_v, sem):
        wid = lax.axis_index("s") * _NC + lax.axis_index("c")
        pltpu.sync_copy(idx_hbm.at[wid], idx_v)
        gathers = []
        for j in range(k):
            gathers.append(pltpu.async_copy(
                ypad_hbm.at[idx_v.at[j]],
                rows_v.at[pl.ds(j * c, c)], sem))
        for g in gathers:
            g.wait()
        pltpu.sync_copy(rows_v, out_hbm.at[pl.ds(wid * per_w, per_w)])

    return kern(y_pad, idx)


def _pass_b_body(x_ref, wt_ref, bt_ref, o_ref):
    zero = jnp.zeros((64, 64), jnp.bfloat16)

    def bd(l):
        rows = []
        for q in range(4):
            pieces = [zero] * 4
            pieces[q] = wt_ref[l, q]
            rows.append(jnp.concatenate(pieces, axis=1))
        return jnp.concatenate(rows, axis=0)

    def bias(l):
        return jnp.concatenate([bt_ref[q, l, :] for q in range(4)])

    x4 = jnp.concatenate(
        [x_ref[q][:, :64] for q in range(4)], axis=1)
    h = x4.astype(jnp.bfloat16)
    for l in range(4):
        acc = jnp.dot(h, bd(l), preferred_element_type=jnp.float32)
        h = jnp.maximum(acc + bias(l), 0.0).astype(jnp.bfloat16)
    y4 = (jnp.dot(h, bd(4), preferred_element_type=jnp.float32)
          + bias(4))
    for q in range(4):
        o_ref[q, :, :64] = y4[:, 64 * q:64 * (q + 1)]
        o_ref[q, :, 64:] = y4[:, 64 * q:64 * (q + 1)]


def _pass_b(x_pad4, wt, bt, tile_rows, n_tiles, h, a):
    nq = n_tiles // 4
    return pl.pallas_call(
        _pass_b_body,
        grid=(nq,),
        in_specs=[
            pl.BlockSpec((4, tile_rows, 128), lambda t: (t, 0, 0)),
            pl.BlockSpec((5, 4, h, h), lambda t: (0, t, 0, 0)),
            pl.BlockSpec((4, 5, a), lambda t: (t, 0, 0)),
        ],
        out_specs=pl.BlockSpec((4, tile_rows, 128), lambda t: (t, 0, 0)),
        out_shape=jax.ShapeDtypeStruct((n_tiles, tile_rows, 128),
                                       jnp.float32),
    )(x_pad4, wt, bt)


def kernel(state, rm_state, W0, b0, W1, b1, W2, b2, W3, b3, W4, b4, W5, b5):
    B, D = state.shape
    E, _, H = W0.shape
    A = W5.shape[2]
    T = 512
    NT = B // T + E
    P = NT * T

    e = rm_state.astype(jnp.int32)
    idx_dst, ts_pad, _ = _sc_route(e, B, T, NT, _NW, _CHUNK)
    tile_start = ts_pad[:E]
    tile_expert = (jnp.sum(
        (jnp.arange(NT, dtype=jnp.int32)[:, None] >= tile_start[None, :])
        .astype(jnp.int32), axis=1) - 1)
    e2d = e.astype(jnp.float32).reshape(B, 1)

    wstk = jnp.stack((W1, W2, W3, W4, W5)).astype(jnp.bfloat16)
    wt = wstk[:, tile_expert]
    bstack = jnp.stack((b1, b2, b3, b4, b5), axis=1)
    bt = bstack[tile_expert]

    b0all = b0.reshape(1, E * H)
    h0sel = _pass_a(state, W0.reshape(E * D, H), b0all, e2d)
    x_pad = _sc_dispatch(h0sel, idx_dst, P)
    x_pad4 = x_pad.reshape(NT, T, 128)
    y_pad4 = _pass_b(x_pad4, wt, bt, T, NT, H, A)
    y_pad = y_pad4.reshape(P, 128)
    wide = _sc_collect(y_pad, idx_dst, B, A)
    return wide[:, :A]


# speedup vs baseline: 1.2375x; 1.0661x over previous
import functools

import jax
import jax.numpy as jnp
from jax import lax
from jax.experimental import pallas as pl
from jax.experimental.pallas import tpu as pltpu
from jax.experimental.pallas import tpu_sc as plsc

_NC = 2
_NS = 16
_NW = _NC * _NS
_CHUNK = 128


def _pass_a_body(w0_ref, x_ref, b_ref, e_ref, o_ref, w_scr):
    i = pl.program_id(0)
    d = w_scr.shape[0]

    @pl.when(i == 0)
    def _():
        w_scr[...] = jnp.concatenate(
            [w0_ref[pl.ds(k * d, d), :] for k in range(8)], axis=1)

    @pl.when(i > 0)
    def _():
        xb = x_ref[...].astype(jnp.bfloat16)
        acc = jnp.dot(xb, w_scr[...], preferred_element_type=jnp.float32)
        h = jnp.maximum(acc + b_ref[...], 0.0)
        ev = e_ref[...].astype(jnp.int32)
        sel = h[:, :64] * (ev == 0)
        for k in range(1, 8):
            sel = sel + h[:, 64 * k:64 * (k + 1)] * (ev == k)
        o_ref[:, :64] = sel
        o_ref[:, 64:] = sel


def _pass_a(state, w0flat, b0all, e2d, block_rows=1024):
    b, d = state.shape
    ed, h = w0flat.shape
    e = ed // d
    eh = e * h

    def shifted(i):
        return jnp.maximum(i - 1, 0)

    return pl.pallas_call(
        _pass_a_body,
        grid=(b // block_rows + 1,),
        in_specs=[
            pl.BlockSpec((ed, h), lambda i: (0, 0)),
            pl.BlockSpec((block_rows, d), lambda i: (shifted(i), 0)),
            pl.BlockSpec((1, eh), lambda i: (0, 0)),
            pl.BlockSpec((block_rows, 1), lambda i: (shifted(i), 0)),
        ],
        out_specs=pl.BlockSpec((block_rows, 128), lambda i: (shifted(i), 0)),
        out_shape=jax.ShapeDtypeStruct((b, 128), jnp.float32),
        scratch_shapes=[pltpu.VMEM((d, eh), jnp.bfloat16)],
    )(w0flat, state, b0all, e2d)


def _sc_route(e_arr, b_rows, t_rows, nt, nw, chunk):
    mesh = plsc.VectorSubcoreMesh(core_axis_name="c", subcore_axis_name="s", num_cores=_NC, num_subcores=_NS)
    per_s = b_rows // _NS
    k = per_s // (2 * chunk)
    nch = per_s // 16

    @functools.partial(
        pl.kernel,
        mesh=mesh,
        compiler_params=pltpu.CompilerParams(needs_layout_passes=False),
        out_type=[
            jax.ShapeDtypeStruct((nw, per_s // (chunk * 2), chunk), jnp.int32),
            jax.ShapeDtypeStruct((32,), jnp.int32),
            jax.ShapeDtypeStruct((_NS, 16), jnp.int32),
        ],
        scratch_types=[
            pltpu.VMEM((per_s,), jnp.int32),
            pltpu.VMEM((per_s,), jnp.int32),
            pltpu.VMEM((16,), jnp.int32),
            pltpu.VMEM((_NS, 16), jnp.int32),
            pltpu.VMEM((16,), jnp.int32),
            pltpu.VMEM((16,), jnp.int32),
            pltpu.VMEM((2, 2, chunk), jnp.int32),
            pltpu.VMEM((32,), jnp.int32),
        ],
    )
    def kern(e_hbm, idst_hbm, te_hbm, cnt_hbm,
             e_v, occ_v, cnt_v, all_v, rsb_v, ts_v, p_v, te_v):
        cid = lax.axis_index("c")
        sid = lax.axis_index("s")

        @pl.when(cid == 0)
        def _():
            iota = lax.iota(jnp.int32, 16)
            pltpu.sync_copy(e_hbm.at[pl.ds(sid * per_s, per_s)], e_v)
            cnt_v[...] = jnp.zeros(16, jnp.int32)
            for ch in range(nch):
                ev = e_v[pl.ds(ch * 16, 16)]
                base = plsc.load_gather(cnt_v, [ev])
                cnt = cnt_v[...]
                within = jnp.zeros(16, jnp.int32)
                for ex in range(8):
                    m = (ev == ex).astype(jnp.int32)
                    pre = plsc.cumsum(m) - m
                    within = within + pre * m
                    tot = jnp.max(plsc.cumsum(m))
                    cnt = cnt + jnp.where(iota == ex, tot, 0)
                occ_v[pl.ds(ch * 16, 16)] = base + within
                cnt_v[...] = cnt
            pltpu.sync_copy(cnt_v, cnt_hbm.at[sid])
            plsc.subcore_barrier()
            pltpu.sync_copy(cnt_hbm, all_v)
            base = jnp.zeros(16, jnp.int32)
            gcnt = jnp.zeros(16, jnp.int32)
            for s in range(_NS):
                row = all_v[s]
                below = jnp.full((16,), s, jnp.int32) < sid
                base = base + jnp.where(below, row, 0)
                gcnt = gcnt + row
            tiles = (gcnt + (t_rows - 1)) // t_rows
            ts_incl = plsc.cumsum(tiles)
            row_start = (ts_incl - tiles) * t_rows
            rsb_v[...] = row_start + base
            ts_v[...] = ts_incl - tiles
            for ch in range(nch):
                ev = e_v[pl.ds(ch * 16, 16)]
                slot = (plsc.load_gather(rsb_v, [ev])
                        + occ_v[pl.ds(ch * 16, 16)])
                p_v[ch // (nch // 2), (ch % (nch // 2)) // (chunk // 16),
                    pl.ds((ch * 16) % chunk, 16)] = slot
            pltpu.sync_copy(p_v, idst_hbm.at[pl.ds(2 * sid, 2)])
            te_v[pl.ds(0, 16)] = ts_v[...]
            pltpu.sync_copy(te_v, te_hbm)

    return kern(e_arr)


def _sc_dispatch(h0sel, idx_dst, p_rows):
    nw, k, c = idx_dst.shape
    per_w = k * c
    width = h0sel.shape[1]
    mesh = plsc.VectorSubcoreMesh(core_axis_name="c", subcore_axis_name="s", num_cores=_NC, num_subcores=_NS)

    @functools.partial(
        pl.kernel,
        mesh=mesh,
        out_type=jax.ShapeDtypeStruct((p_rows, width), jnp.float32),
        scratch_types=[
            pltpu.VMEM((k, c), jnp.int32),
            pltpu.VMEM((per_w, width), jnp.float32),
            pltpu.SemaphoreType.DMA,
        ],
    )
    def kern(h0_hbm, idst_hbm, xpad_hbm, idst_v, rows_v, sem):
        wid = lax.axis_index("s") * _NC + lax.axis_index("c")
        pltpu.sync_copy(idst_hbm.at[wid], idst_v)
        pltpu.sync_copy(h0_hbm.at[pl.ds(wid * per_w, per_w)], rows_v)
        scatters = []
        for j in range(k):
            scatters.append(pltpu.async_copy(
                rows_v.at[pl.ds(j * c, c)],
                xpad_hbm.at[idst_v.at[j]], sem))
        for s in scatters:
            s.wait()

    return kern(h0sel, idx_dst)


def _sc_collect(y_pad, idx, b_rows, a):
    nw, k, c = idx.shape
    per_w = k * c
    width = y_pad.shape[1]
    mesh = plsc.VectorSubcoreMesh(core_axis_name="c", subcore_axis_name="s", num_cores=_NC, num_subcores=_NS)

    @functools.partial(
        pl.kernel,
        mesh=mesh,
        out_type=jax.ShapeDtypeStruct((b_rows, width), jnp.float32),
        scratch_types=[
            pltpu.VMEM((k, c), jnp.int32),
            pltpu.VMEM((per_w, width), jnp.float32),
            pltpu.SemaphoreType.DMA,
        ],
    )
    def kern(ypad_hbm, idx_hbm, out_hbm, idx_v, rows_v, sem):
        wid = lax.axis_index("s") * _NC + lax.axis_index("c")
        pltpu.sync_copy(idx_hbm.at[wid], idx_v)
        gathers = []
        for j in range(k):
            gathers.append(pltpu.async_copy(
                ypad_hbm.at[idx_v.at[j]],
                rows_v.at[pl.ds(j * c, c)], sem))
        for g in gathers:
            g.wait()
        pltpu.sync_copy(rows_v, out_hbm.at[pl.ds(wid * per_w, per_w)])

    return kern(y_pad, idx)


def _pass_b_body(x_ref, wt_ref, bt_ref, o_ref):
    zero = jnp.zeros((64, 64), jnp.bfloat16)

    def bd(l):
        rows = []
        for q in range(4):
            pieces = [zero] * 4
            pieces[q] = wt_ref[l, q]
            rows.append(jnp.concatenate(pieces, axis=1))
        return jnp.concatenate(rows, axis=0)

    def bias(l):
        return jnp.concatenate([bt_ref[q, l, :] for q in range(4)])

    x4 = jnp.concatenate(
        [x_ref[q][:, :64] for q in range(4)], axis=1)
    h = x4.astype(jnp.bfloat16)
    for l in range(4):
        acc = jnp.dot(h, bd(l), preferred_element_type=jnp.float32)
        h = jnp.maximum(acc + bias(l), 0.0).astype(jnp.bfloat16)
    y4 = (jnp.dot(h, bd(4), preferred_element_type=jnp.float32)
          + bias(4))
    for q in range(4):
        o_ref[q, :, :64] = y4[:, 64 * q:64 * (q + 1)]
        o_ref[q, :, 64:] = y4[:, 64 * q:64 * (q + 1)]


def _pass_b(x_pad4, wt, bt, tile_rows, n_tiles, h, a):
    nq = n_tiles // 4
    return pl.pallas_call(
        _pass_b_body,
        grid=(nq,),
        in_specs=[
            pl.BlockSpec((4, tile_rows, 128), lambda t: (t, 0, 0)),
            pl.BlockSpec((5, 4, h, h), lambda t: (0, t, 0, 0)),
            pl.BlockSpec((4, 5, a), lambda t: (t, 0, 0)),
        ],
        out_specs=pl.BlockSpec((4, tile_rows, 128), lambda t: (t, 0, 0)),
        out_shape=jax.ShapeDtypeStruct((n_tiles, tile_rows, 128),
                                       jnp.float32),
    )(x_pad4, wt, bt)


def kernel(state, rm_state, W0, b0, W1, b1, W2, b2, W3, b3, W4, b4, W5, b5):
    B, D = state.shape
    E, _, H = W0.shape
    A = W5.shape[2]
    T = 512
    NT = B // T + E
    P = NT * T

    e = rm_state.astype(jnp.int32)
    idx_dst, ts_pad, _ = _sc_route(e, B, T, NT, _NW, _CHUNK)
    tile_start = ts_pad[:E]
    tile_expert = (jnp.sum(
        (jnp.arange(NT, dtype=jnp.int32)[:, None] >= tile_start[None, :])
        .astype(jnp.int32), axis=1) - 1)
    e2d = rm_state.astype(jnp.int8).reshape(B, 1)

    wstk = jnp.stack((W1, W2, W3, W4, W5)).astype(jnp.bfloat16)
    wt = wstk[:, tile_expert]
    bstack = jnp.stack((b1, b2, b3, b4, b5), axis=1)
    bt = bstack[tile_expert]

    b0all = b0.reshape(1, E * H)
    w0bf = W0.astype(jnp.bfloat16).reshape(E * D, H)
    h0sel = _pass_a(state, w0bf, b0all, e2d)
    x_pad = _sc_dispatch(h0sel, idx_dst, P)
    x_pad4 = x_pad.reshape(NT, T, 128)
    y_pad4 = _pass_b(x_pad4, wt, bt, T, NT, H, A)
    y_pad = y_pad4.reshape(P, 128)
    wide = _sc_collect(y_pad, idx_dst, B, A)
    return wide[:, :A]


# speedup vs baseline: 1.2413x; 1.0031x over previous
import functools

import jax
import jax.numpy as jnp
from jax import lax
from jax.experimental import pallas as pl
from jax.experimental.pallas import tpu as pltpu
from jax.experimental.pallas import tpu_sc as plsc

_NC = 2
_NS = 16
_NW = _NC * _NS
_CHUNK = 128


def _pass_a_body(w0_ref, x_ref, b_ref, e_ref, o_ref, w_scr):
    i = pl.program_id(0)
    d = w_scr.shape[0]

    @pl.when(i == 0)
    def _():
        w_scr[...] = jnp.concatenate(
            [w0_ref[k] for k in range(8)], axis=1)

    @pl.when(i > 0)
    def _():
        xb = x_ref[...].astype(jnp.bfloat16)
        acc = jnp.dot(xb, w_scr[...], preferred_element_type=jnp.float32)
        h = jnp.maximum(acc + b_ref[...], 0.0)
        ev = e_ref[...].astype(jnp.int32)
        sel = h[:, :64] * (ev == 0)
        for k in range(1, 8):
            sel = sel + h[:, 64 * k:64 * (k + 1)] * (ev == k)
        o_ref[:, :64] = sel
        o_ref[:, 64:] = sel


def _pass_a(state, w0bf, b0all, e2d, block_rows=1024):
    b, d = state.shape
    e, _, h = w0bf.shape
    eh = e * h

    def shifted(i):
        return jnp.maximum(i - 1, 0)

    return pl.pallas_call(
        _pass_a_body,
        grid=(b // block_rows + 1,),
        in_specs=[
            pl.BlockSpec((e, d, h), lambda i: (0, 0, 0)),
            pl.BlockSpec((block_rows, d), lambda i: (shifted(i), 0)),
            pl.BlockSpec((1, eh), lambda i: (0, 0)),
            pl.BlockSpec((block_rows, 1), lambda i: (shifted(i), 0)),
        ],
        out_specs=pl.BlockSpec((block_rows, 128), lambda i: (shifted(i), 0)),
        out_shape=jax.ShapeDtypeStruct((b, 128), jnp.float32),
        scratch_shapes=[pltpu.VMEM((d, eh), jnp.bfloat16)],
    )(w0bf, state, b0all, e2d)


def _sc_route(e_arr, b_rows, t_rows, nt, nw, chunk):
    mesh = plsc.VectorSubcoreMesh(core_axis_name="c", subcore_axis_name="s", num_cores=_NC, num_subcores=_NS)
    per_s = b_rows // _NS
    k = per_s // (2 * chunk)
    nch = per_s // 16

    @functools.partial(
        pl.kernel,
        mesh=mesh,
        compiler_params=pltpu.CompilerParams(needs_layout_passes=False),
        out_type=[
            jax.ShapeDtypeStruct((nw, per_s // (chunk * 2), chunk), jnp.int32),
            jax.ShapeDtypeStruct((32,), jnp.int32),
            jax.ShapeDtypeStruct((_NS, 16), jnp.int32),
        ],
        scratch_types=[
            pltpu.VMEM((per_s,), jnp.int32),
            pltpu.VMEM((per_s,), jnp.int32),
            pltpu.VMEM((16,), jnp.int32),
            pltpu.VMEM((_NS, 16), jnp.int32),
            pltpu.VMEM((16,), jnp.int32),
            pltpu.VMEM((16,), jnp.int32),
            pltpu.VMEM((2, 2, chunk), jnp.int32),
            pltpu.VMEM((32,), jnp.int32),
        ],
    )
    def kern(e_hbm, idst_hbm, te_hbm, cnt_hbm,
             e_v, occ_v, cnt_v, all_v, rsb_v, ts_v, p_v, te_v):
        cid = lax.axis_index("c")
        sid = lax.axis_index("s")

        @pl.when(cid == 0)
        def _():
            iota = lax.iota(jnp.int32, 16)
            pltpu.sync_copy(e_hbm.at[pl.ds(sid * per_s, per_s)], e_v)
            cnt_v[...] = jnp.zeros(16, jnp.int32)
            for ch in range(nch):
                ev = e_v[pl.ds(ch * 16, 16)]
                base = plsc.load_gather(cnt_v, [ev])
                cnt = cnt_v[...]
                within = jnp.zeros(16, jnp.int32)
                for ex in range(8):
                    m = (ev == ex).astype(jnp.int32)
                    pre = plsc.cumsum(m) - m
                    within = within + pre * m
                    tot = jnp.max(plsc.cumsum(m))
                    cnt = cnt + jnp.where(iota == ex, tot, 0)
                occ_v[pl.ds(ch * 16, 16)] = base + within
                cnt_v[...] = cnt
            pltpu.sync_copy(cnt_v, cnt_hbm.at[sid])
            plsc.subcore_barrier()
            pltpu.sync_copy(cnt_hbm, all_v)
            base = jnp.zeros(16, jnp.int32)
            gcnt = jnp.zeros(16, jnp.int32)
            for s in range(_NS):
                row = all_v[s]
                below = jnp.full((16,), s, jnp.int32) < sid
                base = base + jnp.where(below, row, 0)
                gcnt = gcnt + row
            tiles = (gcnt + (t_rows - 1)) // t_rows
            ts_incl = plsc.cumsum(tiles)
            row_start = (ts_incl - tiles) * t_rows
            rsb_v[...] = row_start + base
            ts_v[...] = ts_incl - tiles
            for ch in range(nch):
                ev = e_v[pl.ds(ch * 16, 16)]
                slot = (plsc.load_gather(rsb_v, [ev])
                        + occ_v[pl.ds(ch * 16, 16)])
                p_v[ch // (nch // 2), (ch % (nch // 2)) // (chunk // 16),
                    pl.ds((ch * 16) % chunk, 16)] = slot
            pltpu.sync_copy(p_v, idst_hbm.at[pl.ds(2 * sid, 2)])
            te_v[pl.ds(0, 16)] = ts_v[...]
            pltpu.sync_copy(te_v, te_hbm)

    return kern(e_arr)


def _sc_dispatch(h0sel, idx_dst, p_rows):
    nw, k, c = idx_dst.shape
    per_w = k * c
    width = h0sel.shape[1]
    mesh = plsc.VectorSubcoreMesh(core_axis_name="c", subcore_axis_name="s", num_cores=_NC, num_subcores=_NS)

    @functools.partial(
        pl.kernel,
        mesh=mesh,
        out_type=jax.ShapeDtypeStruct((p_rows, width), jnp.float32),
        scratch_types=[
            pltpu.VMEM((k, c), jnp.int32),
            pltpu.VMEM((per_w, width), jnp.float32),
            pltpu.SemaphoreType.DMA,
        ],
    )
    def kern(h0_hbm, idst_hbm, xpad_hbm, idst_v, rows_v, sem):
        wid = lax.axis_index("s") * _NC + lax.axis_index("c")
        pltpu.sync_copy(idst_hbm.at[wid], idst_v)
        pltpu.sync_copy(h0_hbm.at[pl.ds(wid * per_w, per_w)], rows_v)
        scatters = []
        for j in range(k):
            scatters.append(pltpu.async_copy(
                rows_v.at[pl.ds(j * c, c)],
                xpad_hbm.at[idst_v.at[j]], sem))
        for s in scatters:
            s.wait()

    return kern(h0sel, idx_dst)


def _sc_collect(y_pad, idx, b_rows, a):
    nw, k, c = idx.shape
    per_w = k * c
    width = y_pad.shape[1]
    mesh = plsc.VectorSubcoreMesh(core_axis_name="c", subcore_axis_name="s", num_cores=_NC, num_subcores=_NS)

    @functools.partial(
        pl.kernel,
        mesh=mesh,
        out_type=jax.ShapeDtypeStruct((b_rows, width), jnp.float32),
        scratch_types=[
            pltpu.VMEM((k, c), jnp.int32),
            pltpu.VMEM((per_w, width), jnp.float32),
            pltpu.SemaphoreType.DMA,
        ],
    )
    def kern(ypad_hbm, idx_hbm, out_hbm, idx_v, rows_v, sem):
        wid = lax.axis_index("s") * _NC + lax.axis_index("c")
        pltpu.sync_copy(idx_hbm.at[wid], idx_v)
        gathers = []
        for j in range(k):
            gathers.append(pltpu.async_copy(
                ypad_hbm.at[idx_v.at[j]],
                rows_v.at[pl.ds(j * c, c)], sem))
        for g in gathers:
            g.wait()
        pltpu.sync_copy(rows_v, out_hbm.at[pl.ds(wid * per_w, per_w)])

    return kern(y_pad, idx)


def _pass_b_body(x_ref, wt_ref, bt_ref, o_ref):
    zero = jnp.zeros((64, 64), jnp.bfloat16)

    def bd(l):
        rows = []
        for q in range(4):
            pieces = [zero] * 4
            pieces[q] = wt_ref[l, q]
            rows.append(jnp.concatenate(pieces, axis=1))
        return jnp.concatenate(rows, axis=0)

    def bias(l):
        return jnp.concatenate([bt_ref[q, l, :] for q in range(4)])

    x4 = jnp.concatenate(
        [x_ref[q][:, :64] for q in range(4)], axis=1)
    h = x4.astype(jnp.bfloat16)
    for l in range(4):
        acc = jnp.dot(h, bd(l), preferred_element_type=jnp.float32)
        h = jnp.maximum(acc + bias(l), 0.0).astype(jnp.bfloat16)
    y4 = (jnp.dot(h, bd(4), preferred_element_type=jnp.float32)
          + bias(4))
    for q in range(4):
        o_ref[q, :, :64] = y4[:, 64 * q:64 * (q + 1)]
        o_ref[q, :, 64:] = y4[:, 64 * q:64 * (q + 1)]


def _pass_b(x_pad4, wt, bt, tile_rows, n_tiles, h, a):
    nq = n_tiles // 4
    return pl.pallas_call(
        _pass_b_body,
        grid=(nq,),
        in_specs=[
            pl.BlockSpec((4, tile_rows, 128), lambda t: (t, 0, 0)),
            pl.BlockSpec((5, 4, h, h), lambda t: (0, t, 0, 0)),
            pl.BlockSpec((4, 5, a), lambda t: (t, 0, 0)),
        ],
        out_specs=pl.BlockSpec((4, tile_rows, 128), lambda t: (t, 0, 0)),
        out_shape=jax.ShapeDtypeStruct((n_tiles, tile_rows, 128),
                                       jnp.float32),
    )(x_pad4, wt, bt)


def kernel(state, rm_state, W0, b0, W1, b1, W2, b2, W3, b3, W4, b4, W5, b5):
    B, D = state.shape
    E, _, H = W0.shape
    A = W5.shape[2]
    T = 512
    NT = B // T + E
    P = NT * T

    e = rm_state.astype(jnp.int32)
    idx_dst, ts_pad, _ = _sc_route(e, B, T, NT, _NW, _CHUNK)
    tile_start = ts_pad[:E]
    tile_expert = (jnp.sum(
        (jnp.arange(NT, dtype=jnp.int32)[:, None] >= tile_start[None, :])
        .astype(jnp.int32), axis=1) - 1)
    e2d = rm_state.astype(jnp.int8).reshape(B, 1)

    wstk = jnp.stack((W1, W2, W3, W4, W5)).astype(jnp.bfloat16)
    wt = wstk[:, tile_expert]
    bstack = jnp.stack((b1, b2, b3, b4, b5), axis=1)
    bt = bstack[tile_expert]

    b0all = b0.reshape(1, E * H)
    w0bf = W0.astype(jnp.bfloat16)
    h0sel = _pass_a(state, w0bf, b0all, e2d)
    x_pad = _sc_dispatch(h0sel, idx_dst, P)
    x_pad4 = x_pad.reshape(NT, T, 128)
    y_pad4 = _pass_b(x_pad4, wt, bt, T, NT, H, A)
    y_pad = y_pad4.reshape(P, 128)
    wide = _sc_collect(y_pad, idx_dst, B, A)
    return wide[:, :A]
